# Initial kernel scaffold; baseline (speedup 1.0000x reference)
import jax
import jax.numpy as jnp
from jax.experimental import pallas as pl


def kernel(mol_x, mol_x_feat, bond_x, atom_edge_index, clique_x, clique_edge_index, atom2clique_index, residue_x, residue_evo_x, residue_edge_index, residue_edge_weight, mol_batch, params):
    raise NotImplementedError("write your pallas kernel here")


# speedup vs baseline: 1.0037x; 1.0037x over previous
import functools

import jax
import jax.numpy as jnp
import numpy as np
from jax.experimental import pallas as pl

N = 50000
E = 800000
H = 64
T = 2
FT = H // T
G = 512
MOL_DEG = np.array([0, 1200, 5400, 12000, 18000, 9000, 3200, 1200], dtype=np.float64)
_bins = np.arange(len(MOL_DEG))
AVG_LOG = float((np.log(_bins + 1.0) * MOL_DEG).sum() / MOL_DEG.sum())


def _layer_norm(x, w, b):
    mu = x.mean(-1, keepdims=True)
    var = ((x - mu) ** 2).mean(-1, keepdims=True)
    return (x - mu) / jnp.sqrt(var + 1e-5) * w + b


def _mlp2(x, m, out_norm=False):
    x = jax.nn.relu(x @ m['W0'] + m['b0'])
    x = x @ m['W1'] + m['b1']
    if out_norm:
        x = _layer_norm(x, m['ln_w'], m['ln_b'])
    return x


def _pna_conv(x, edge_attr, edge_index, lyr):
    src = edge_index[0]
    dst = edge_index[1]
    e = edge_attr @ lyr['edge_W'] + lyr['edge_b']
    ones = jnp.ones((src.shape[0],), jnp.float32)
    cnt = jax.ops.segment_sum(ones, dst, N)
    deg = jnp.maximum(cnt, 1.0)
    amp = jnp.log(deg + 1.0) / AVG_LOG
    att = AVG_LOG / jnp.log(deg + 1.0)
    has = (cnt > 0)[:, None]
    outs = []
    for t in range(T):
        xt = x[:, t * FT:(t + 1) * FT]
        h = jnp.concatenate([xt[dst], xt[src], e], axis=-1)
        m = _mlp2(h, lyr['pre'][t])
        s = jax.ops.segment_sum(m, dst, N)
        mean = s / deg[:, None]
        mn = jnp.where(has, jax.ops.segment_min(m, dst, N), 0.0)
        mx = jnp.where(has, jax.ops.segment_max(m, dst, N), 0.0)
        sq = jax.ops.segment_sum(m * m, dst, N) / deg[:, None]
        std = jnp.sqrt(jax.nn.relu(sq - mean ** 2) + 1e-5)
        aggr = jnp.concatenate([mean, mn, mx, std], axis=-1)
        scaled = jnp.concatenate([aggr, aggr * amp[:, None], aggr * att[:, None]], axis=-1)
        outs.append(_mlp2(jnp.concatenate([xt, scaled], axis=-1), lyr['post'][t]))
    return jnp.concatenate(outs, axis=-1) @ lyr['lin_W'] + lyr['lin_b']


def _graph_norm(x, batch, lyr):
    cnt = jnp.maximum(jax.ops.segment_sum(jnp.ones((N,), jnp.float32), batch, G), 1.0)
    mean = jax.ops.segment_sum(x, batch, G) / cnt[:, None]
    out = x - lyr['gn_ms'] * mean[batch]
    var = jax.ops.segment_sum(out * out, batch, G) / cnt[:, None]
    std = jnp.sqrt(var + 1e-5)
    return lyr['gn_w'] * out / std[batch] + lyr['gn_b']


def _reg_kernel(pool_ref, w0_ref, b0_ref, w1_ref, b1_ref, out_ref):
    h = jnp.maximum(pool_ref[...] @ w0_ref[...] + b0_ref[...], 0.0)
    out_ref[...] = h @ w1_ref[...] + b1_ref[...]


def _reg_head(pool, reg):
    return pl.pallas_call(
        _reg_kernel,
        out_shape=jax.ShapeDtypeStruct((G, 1), jnp.float32),
    )(pool, reg['W0'], reg['b0'][None, :], reg['W1'], reg['b1'][None, :])


def kernel(mol_x, mol_x_feat, bond_x, atom_edge_index, clique_x, clique_edge_index, atom2clique_index, residue_x, residue_evo_x, residue_edge_index, residue_edge_weight, mol_batch, params):
    atom_x = params['emb'][mol_x] + _mlp2(mol_x_feat, params['feat'], out_norm=True)
    for lyr in params['layers']:
        atom_x = _pna_conv(atom_x, bond_x, atom_edge_index, lyr)
        atom_x = _graph_norm(atom_x, mol_batch, lyr)
    cnt = jnp.maximum(jax.ops.segment_sum(jnp.ones((N,), jnp.float32), mol_batch, G), 1.0)
    pool = jax.ops.segment_sum(atom_x, mol_batch, G) / cnt[:, None]
    return _reg_head(pool, params['reg'])


# speedup vs baseline: 2.4030x; 2.3942x over previous
import functools

import jax
import jax.numpy as jnp
import numpy as np
from jax import lax
from jax.experimental import pallas as pl
from jax.experimental.pallas import tpu as pltpu
from jax.experimental.pallas import tpu_sc as plsc

N = 50000
E = 800000
H = 64
T = 2
FT = H // T
G = 512
MOL_DEG = np.array([0, 1200, 5400, 12000, 18000, 9000, 3200, 1200], dtype=np.float64)
_bins = np.arange(len(MOL_DEG))
AVG_LOG = float((np.log(_bins + 1.0) * MOL_DEG).sum() / MOL_DEG.sum())

NT = 32
CHUNK = 128
CPT = 196
EPAD = NT * CPT * CHUNK
RCH = 128
ACC_N = 449
NPH = 4
GACC = 49
NPG = 50176

_half = [0, 25344, 50688]
while _half[-1] + 24960 < 400000:
    _half.append(_half[-1] + 24960)
EW = np.array(_half + [400000] + [400000 + b for b in _half[1:]] + [800000],
              dtype=np.int32)
GW = np.array([3200 * k for k in range(14)] + [44480, 47360, 50000],
              dtype=np.int32)

_sc_mesh = plsc.VectorSubcoreMesh(core_axis_name="c", subcore_axis_name="s")
_sc_params = pltpu.CompilerParams(use_tc_tiling_on_sc=False)


def _wid():
    return lax.axis_index("s") * 2 + lax.axis_index("c")


@functools.partial(
    pl.kernel,
    mesh=_sc_mesh,
    compiler_params=_sc_params,
    out_type=jax.ShapeDtypeStruct((EPAD, H), jnp.float32),
    scratch_types=[
        pltpu.VMEM((CPT, CHUNK), jnp.int32),
        pltpu.VMEM((3, CHUNK, H), jnp.float32),
        pltpu.SemaphoreType.DMA,
        pltpu.SemaphoreType.DMA((3,)),
        pltpu.SemaphoreType.DMA((3,)),
    ],
)
def _sc_permute(idx_hbm, table_hbm, out_hbm, idx_v, buf, sem_i, sem_g, sem_o):
    w = _wid()
    base = w * (CPT * CHUNK)
    pltpu.async_copy(idx_hbm.at[w], idx_v, sem_i).wait()

    def start_g(j, p):
        pltpu.async_copy(table_hbm.at[idx_v.at[j]], buf.at[p], sem_g.at[p])

    def wait_g(p):
        pltpu.make_async_copy(table_hbm.at[idx_v.at[0]], buf.at[p], sem_g.at[p]).wait()

    def start_o(j, p):
        pltpu.async_copy(buf.at[p], out_hbm.at[pl.ds(base + j * CHUNK, CHUNK), :],
                         sem_o.at[p])

    def wait_o(p):
        pltpu.make_async_copy(buf.at[p], out_hbm.at[pl.ds(0, CHUNK), :],
                              sem_o.at[p]).wait()

    start_g(0, 0)
    start_g(1, 1)

    def body(j, carry):
        p = j % 3

        @pl.when(j >= 1)
        def _():
            wait_o((j - 1) % 3)

        @pl.when(j + 2 < CPT)
        def _():
            start_g(j + 2, (j + 2) % 3)

        wait_g(p)
        start_o(j, p)
        return carry

    lax.fori_loop(0, CPT, body, 0)
    wait_o((CPT - 1) % 3)


@functools.partial(
    pl.kernel,
    mesh=_sc_mesh,
    compiler_params=_sc_params,
    out_type=[jax.ShapeDtypeStruct((EPAD, H), jnp.float32),
              jax.ShapeDtypeStruct((EPAD, H), jnp.float32)],
    scratch_types=[
        pltpu.VMEM((CPT, CHUNK), jnp.int32),
        pltpu.VMEM((CPT, CHUNK), jnp.int32),
        pltpu.VMEM((3, CHUNK, H), jnp.float32),
        pltpu.VMEM((3, CHUNK, H), jnp.float32),
        pltpu.SemaphoreType.DMA,
        pltpu.SemaphoreType.DMA((3,)),
        pltpu.SemaphoreType.DMA((3,)),
        pltpu.SemaphoreType.DMA((3,)),
        pltpu.SemaphoreType.DMA((3,)),
    ],
)
def _sc_gather2(dst_hbm, src_hbm, x_hbm, o1_hbm, o2_hbm,
                di_v, si_v, buf1, buf2, sem_i, sem_a, sem_b, sem_o1, sem_o2):
    w = _wid()
    base = w * (CPT * CHUNK)
    pltpu.async_copy(dst_hbm.at[w], di_v, sem_i).wait()
    pltpu.async_copy(src_hbm.at[w], si_v, sem_i).wait()

    def start_g(j, p):
        pltpu.async_copy(x_hbm.at[di_v.at[j]], buf1.at[p], sem_a.at[p])
        pltpu.async_copy(x_hbm.at[si_v.at[j]], buf2.at[p], sem_b.at[p])

    def wait_g(p):
        pltpu.make_async_copy(x_hbm.at[di_v.at[0]], buf1.at[p], sem_a.at[p]).wait()
        pltpu.make_async_copy(x_hbm.at[si_v.at[0]], buf2.at[p], sem_b.at[p]).wait()

    def start_o(j, p):
        sl = out_slice = pl.ds(base + j * CHUNK, CHUNK)
        pltpu.async_copy(buf1.at[p], o1_hbm.at[sl, :], sem_o1.at[p])
        pltpu.async_copy(buf2.at[p], o2_hbm.at[sl, :], sem_o2.at[p])

    def wait_o(p):
        pltpu.make_async_copy(buf1.at[p], o1_hbm.at[pl.ds(0, CHUNK), :],
                              sem_o1.at[p]).wait()
        pltpu.make_async_copy(buf2.at[p], o2_hbm.at[pl.ds(0, CHUNK), :],
                              sem_o2.at[p]).wait()

    start_g(0, 0)
    start_g(1, 1)

    def body(j, carry):
        p = j % 3

        @pl.when(j >= 1)
        def _():
            wait_o((j - 1) % 3)

        @pl.when(j + 2 < CPT)
        def _():
            start_g(j + 2, (j + 2) % 3)

        wait_g(p)
        start_o(j, p)
        return carry

    lax.fori_loop(0, CPT, body, 0)
    wait_o((CPT - 1) % 3)


@functools.partial(
    pl.kernel,
    mesh=_sc_mesh,
    compiler_params=_sc_params,
    out_type=jax.ShapeDtypeStruct((T, NT * NPH, ACC_N, 4 * FT), jnp.float32),
    scratch_types=[
        pltpu.VMEM((NT, 16), jnp.int32),
        pltpu.VMEM((NT, 16), jnp.int32),
        pltpu.VMEM((ACC_N, 4 * FT), jnp.float32),
        pltpu.VMEM((2, RCH, FT), jnp.float32),
        pltpu.VMEM((2, RCH), jnp.int32),
        pltpu.SemaphoreType.DMA,
        pltpu.SemaphoreType.DMA((2,)),
        pltpu.SemaphoreType.DMA((2,)),
        pltpu.SemaphoreType.DMA,
    ],
)
def _sc_edge_reduce(eb_hbm, nb_hbm, m_hbm, dst_hbm, out_hbm,
                    eb_v, nb_v, acc, mbuf, dbuf, sem_i, sem_m, sem_d, sem_w):
    w = _wid()
    pltpu.async_copy(eb_hbm, eb_v, sem_i).wait()
    pltpu.async_copy(nb_hbm, nb_v, sem_i).wait()
    ebv = eb_v[w, pl.ds(0, 16)]
    nbv = nb_v[w, pl.ds(0, 16)]
    zeros = jnp.zeros((16,), jnp.float32)
    pinf = jnp.full((16,), jnp.inf, jnp.float32)
    ninf = jnp.full((16,), -jnp.inf, jnp.float32)

    for tw in range(T):
        for ph in range(NPH):
            e0 = ebv[ph]
            e1 = ebv[ph + 1]
            nb = nbv[ph]
            e0a = (e0 // 8) * 8
            nch = (e1 - e0a + (RCH - 1)) // RCH

            def initrow(r, c):
                for k in range(2 * FT // 16):
                    acc[r, pl.ds(k * 16, 16)] = zeros
                for k in range(FT // 16):
                    acc[r, pl.ds(2 * FT + k * 16, 16)] = pinf
                for k in range(FT // 16):
                    acc[r, pl.ds(3 * FT + k * 16, 16)] = ninf
                return c

            lax.fori_loop(0, ACC_N, initrow, 0)

            def start_in(ch, p):
                eb = e0a + ch * RCH
                pltpu.async_copy(
                    m_hbm.at[pl.ds(eb, RCH), pl.ds(tw * FT, FT)],
                    mbuf.at[p], sem_m.at[p])
                pltpu.async_copy(dst_hbm.at[pl.ds(eb, RCH)], dbuf.at[p],
                                 sem_d.at[p])

            def wait_in(p):
                pltpu.make_async_copy(
                    m_hbm.at[pl.ds(0, RCH), pl.ds(0, FT)], mbuf.at[p],
                    sem_m.at[p]).wait()
                pltpu.make_async_copy(dst_hbm.at[pl.ds(0, RCH)], dbuf.at[p],
                                      sem_d.at[p]).wait()

            @pl.when(nch > 0)
            def _():
                start_in(0, 0)

            @pl.when(nch > 1)
            def _():
                start_in(1, 1)

            def chunk(ch, c):
                p = ch % 2
                wait_in(p)
                ebase = e0a + ch * RCH

                def edge16(i, c2):
                    jb = i * 16
                    dvec = dbuf[p, pl.ds(jb, 16)] - nb
                    ge_v = lax.iota(jnp.int32, 16) + (ebase + jb)
                    okv = jnp.logical_and(ge_v >= e0, ge_v < e1)
                    rowv = jnp.where(okv, dvec, ACC_N - 1)
                    for k in range(16):
                        row = rowv[k]
                        for f in range(FT // 16):
                            sl = pl.ds(f * 16, 16)
                            v = mbuf[p, jb + k, sl]
                            acc[row, pl.ds(f * 16, 16)] = (
                                acc[row, pl.ds(f * 16, 16)] + v)
                            acc[row, pl.ds(FT + f * 16, 16)] = (
                                acc[row, pl.ds(FT + f * 16, 16)] + v * v)
                            acc[row, pl.ds(2 * FT + f * 16, 16)] = jnp.minimum(
                                acc[row, pl.ds(2 * FT + f * 16, 16)], v)
                            acc[row, pl.ds(3 * FT + f * 16, 16)] = jnp.maximum(
                                acc[row, pl.ds(3 * FT + f * 16, 16)], v)
                    return c2

                lax.fori_loop(0, RCH // 16, edge16, 0)

                @pl.when(ch + 2 < nch)
                def _():
                    start_in(ch + 2, p)

                return c

            lax.fori_loop(0, nch, chunk, 0)
            pltpu.async_copy(acc, out_hbm.at[tw, w * NPH + ph], sem_w).wait()


@functools.partial(
    pl.kernel,
    mesh=_sc_mesh,
    compiler_params=_sc_params,
    out_type=jax.ShapeDtypeStruct((16, GACC, H), jnp.float32),
    scratch_types=[
        pltpu.VMEM((16, 16), jnp.int32),
        pltpu.VMEM((GACC, H), jnp.float32),
        pltpu.VMEM((2, RCH, H), jnp.float32),
        pltpu.VMEM((2, RCH), jnp.int32),
        pltpu.SemaphoreType.DMA,
        pltpu.SemaphoreType.DMA((2,)),
        pltpu.SemaphoreType.DMA((2,)),
        pltpu.SemaphoreType.DMA,
    ],
)
def _sc_graph_sum(gb_hbm, x_hbm, b_hbm, out_hbm,
                  gb_v, acc, xbuf, bbuf, sem_i, sem_x, sem_b, sem_w):
    w = _wid()

    @pl.when(w < 16)
    def _():
        pltpu.async_copy(gb_hbm, gb_v, sem_i).wait()
        gbv = gb_v[w, pl.ds(0, 16)]
        e0 = gbv[0]
        e1 = gbv[1]
        gb = gbv[2]
        zeros = jnp.zeros((16,), jnp.float32)
        e0a = (e0 // 8) * 8
        nch = (e1 - e0a + (RCH - 1)) // RCH

        def initrow(r, c):
            for k in range(H // 16):
                acc[r, pl.ds(k * 16, 16)] = zeros
            return c

        lax.fori_loop(0, GACC, initrow, 0)

        def start_in(ch, p):
            eb = e0a + ch * RCH
            pltpu.async_copy(x_hbm.at[pl.ds(eb, RCH), :], xbuf.at[p], sem_x.at[p])
            pltpu.async_copy(b_hbm.at[pl.ds(eb, RCH)], bbuf.at[p], sem_b.at[p])

        def wait_in(p):
            pltpu.make_async_copy(x_hbm.at[pl.ds(0, RCH), :], xbuf.at[p],
                                  sem_x.at[p]).wait()
            pltpu.make_async_copy(b_hbm.at[pl.ds(0, RCH)], bbuf.at[p],
                                  sem_b.at[p]).wait()

        @pl.when(nch > 0)
        def _():
            start_in(0, 0)

        @pl.when(nch > 1)
        def _():
            start_in(1, 1)

        def chunk(ch, c):
            p = ch % 2
            wait_in(p)
            ebase = e0a + ch * RCH

            def row16(i, c2):
                jb = i * 16
                bvec = bbuf[p, pl.ds(jb, 16)] - gb
                ge_v = lax.iota(jnp.int32, 16) + (ebase + jb)
                okv = jnp.logical_and(ge_v >= e0, ge_v < e1)
                rowv = jnp.where(okv, bvec, GACC - 1)
                for k in range(16):
                    row = rowv[k]
                    for f in range(H // 16):
                        sl = pl.ds(f * 16, 16)
                        acc[row, sl] = acc[row, sl] + xbuf[p, jb + k, sl]
                return c2

            lax.fori_loop(0, RCH // 16, row16, 0)

            @pl.when(ch + 2 < nch)
            def _():
                start_in(ch + 2, p)

            return c

        lax.fori_loop(0, nch, chunk, 0)
        pltpu.async_copy(acc, out_hbm.at[w], sem_w).wait()


_BT = 1024


def _edge_mlp_kernel(bond_ref, g1_ref, g2_ref, ew_ref, eb_ref,
                     w00_ref, b00_ref, w10_ref, b10_ref,
                     w01_ref, b01_ref, w11_ref, b11_ref, out_ref):
    e = bond_ref[...] @ ew_ref[...] + eb_ref[...]
    outs = []
    for t, (w0, b0, w1, b1) in enumerate((
            (w00_ref, b00_ref, w10_ref, b10_ref),
            (w01_ref, b01_ref, w11_ref, b11_ref))):
        h = jnp.concatenate(
            [g1_ref[:, t * FT:(t + 1) * FT], g2_ref[:, t * FT:(t + 1) * FT], e],
            axis=1)
        pre = jnp.maximum(h @ w0[...] + b0[...], 0.0)
        outs.append(pre @ w1[...] + b1[...])
    out_ref[...] = jnp.concatenate(outs, axis=1)


def _edge_mlp(bond_p, g1, g2, lyr):
    grid = (EPAD // _BT,)
    bspec = pl.BlockSpec((_BT, H), lambda i: (i, 0))
    wspec = lambda shape: pl.BlockSpec(shape, lambda i: (0,) * len(shape))
    return pl.pallas_call(
        _edge_mlp_kernel,
        grid=grid,
        in_specs=[bspec, bspec, bspec,
                  wspec((H, FT)), wspec((1, FT)),
                  wspec((3 * FT, FT)), wspec((1, FT)),
                  wspec((FT, FT)), wspec((1, FT)),
                  wspec((3 * FT, FT)), wspec((1, FT)),
                  wspec((FT, FT)), wspec((1, FT))],
        out_specs=pl.BlockSpec((_BT, H), lambda i: (i, 0)),
        out_shape=jax.ShapeDtypeStruct((EPAD, H), jnp.float32),
    )(bond_p, g1, g2,
      lyr['edge_W'], lyr['edge_b'][None, :],
      lyr['pre'][0]['W0'], lyr['pre'][0]['b0'][None, :],
      lyr['pre'][0]['W1'], lyr['pre'][0]['b1'][None, :],
      lyr['pre'][1]['W0'], lyr['pre'][1]['b0'][None, :],
      lyr['pre'][1]['W1'], lyr['pre'][1]['b1'][None, :])


def _reg_kernel(pool_ref, w0_ref, b0_ref, w1_ref, b1_ref, out_ref):
    h = jnp.maximum(pool_ref[...] @ w0_ref[...] + b0_ref[...], 0.0)
    out_ref[...] = h @ w1_ref[...] + b1_ref[...]


def _reg_head(pool, reg):
    return pl.pallas_call(
        _reg_kernel,
        out_shape=jax.ShapeDtypeStruct((G, 1), jnp.float32),
    )(pool, reg['W0'], reg['b0'][None, :], reg['W1'], reg['b1'][None, :])


def _layer_norm(x, wt, b):
    mu = x.mean(-1, keepdims=True)
    var = ((x - mu) ** 2).mean(-1, keepdims=True)
    return (x - mu) / jnp.sqrt(var + 1e-5) * wt + b


def _mlp2(x, m, out_norm=False):
    x = jax.nn.relu(x @ m['W0'] + m['b0'])
    x = x @ m['W1'] + m['b1']
    if out_norm:
        x = _layer_norm(x, m['ln_w'], m['ln_b'])
    return x


def _combine2(left, right, two, valid1, op, fill):
    both = op(left, right)
    return jnp.where(two, both, jnp.where(valid1, left, fill))


def _graph_segment_sum(x, gbounds, gmeta):
    xp = jnp.concatenate([x, jnp.zeros((NPG - N, H), x.dtype)], axis=0)
    slab = _sc_graph_sum(gbounds, xp, gmeta['batch_pad'])
    flat = slab.reshape(16 * GACC, H)
    left = flat[gmeta['gi1']]
    right = flat[gmeta['gi2']]
    return _combine2(left, right, gmeta['gtwo'][:, None],
                     gmeta['gvalid1'][:, None], jnp.add, 0.0)


def kernel(mol_x, mol_x_feat, bond_x, atom_edge_index, clique_x, clique_edge_index, atom2clique_index, residue_x, residue_evo_x, residue_edge_index, residue_edge_weight, mol_batch, params):
    src = atom_edge_index[0]
    dst = atom_edge_index[1]

    perm = jnp.argsort(dst, stable=True)
    dst_s = dst[perm].astype(jnp.int32)
    src_s = src[perm].astype(jnp.int32)
    row_start = jnp.searchsorted(dst_s, jnp.arange(N + 1), side='left').astype(jnp.int32)
    cnt = (row_start[1:] - row_start[:-1]).astype(jnp.float32)

    ew = jnp.asarray(EW)
    nlo = dst_s[ew[:-1]]
    nhi = dst_s[ew[1:] - 1]
    qnodes = []
    for k in range(NPH + 1):
        qnodes.append(nlo + ((nhi + 1 - nlo) * k) // NPH)
    qn = jnp.stack(qnodes, axis=1)
    qe = jnp.clip(row_start[qn], ew[:-1, None], ew[1:, None])
    qe = qe.at[:, 0].set(ew[:-1]).at[:, NPH].set(ew[1:])
    ebounds = jnp.zeros((NT, 16), jnp.int32).at[:, :NPH + 1].set(qe)
    nbases = jnp.zeros((NT, 16), jnp.int32).at[:, :NPH].set(qn[:, :NPH])

    pe = qe[:, :NPH].reshape(-1)
    nb_flat = qn[:, :NPH].reshape(-1)
    rs = row_start[:N]
    re = row_start[1:]
    p1 = jnp.clip(jnp.searchsorted(pe, rs, side='right') - 1, 0, NT * NPH - 1)
    p2 = jnp.clip(jnp.searchsorted(pe, re - 1, side='right') - 1, 0, NT * NPH - 1)
    off1 = jnp.arange(N) - nb_flat[p1]
    off2 = jnp.arange(N) - nb_flat[p2]
    valid1 = (off1 >= 0) & (off1 < ACC_N - 1)
    two = (p2 > p1) & (off2 >= 0) & (off2 < ACC_N - 1)
    ei1 = p1 * ACC_N + jnp.clip(off1, 0, ACC_N - 1)
    ei2 = p2 * ACC_N + jnp.clip(off2, 0, ACC_N - 1)

    dst_g = jnp.concatenate([dst_s, jnp.zeros((EPAD - E,), jnp.int32)])
    src_g = jnp.concatenate([src_s, jnp.zeros((EPAD - E,), jnp.int32)])
    perm_g = jnp.concatenate([perm.astype(jnp.int32), jnp.zeros((EPAD - E,), jnp.int32)])
    dst3 = dst_g.reshape(NT, CPT, CHUNK)
    src3 = src_g.reshape(NT, CPT, CHUNK)
    perm3 = perm_g.reshape(NT, CPT, CHUNK)

    bond_p = _sc_permute(perm3, bond_x)

    batch = mol_batch.astype(jnp.int32)
    row_g = jnp.searchsorted(batch, jnp.arange(G + 1), side='left').astype(jnp.int32)
    cntb = jnp.maximum((row_g[1:] - row_g[:-1]).astype(jnp.float32), 1.0)
    gw = jnp.asarray(GW)
    gfirst = batch[jnp.minimum(gw[:-1], N - 1)]
    gbounds = jnp.zeros((16, 16), jnp.int32)
    gbounds = gbounds.at[:, 0].set(gw[:-1]).at[:, 1].set(gw[1:]).at[:, 2].set(gfirst)
    grs = row_g[:G]
    gre = row_g[1:]
    gp1 = jnp.clip(jnp.searchsorted(gw[:-1], grs, side='right') - 1, 0, 15)
    gp2 = jnp.clip(jnp.searchsorted(gw[:-1], gre - 1, side='right') - 1, 0, 15)
    goff1 = jnp.arange(G) - gfirst[gp1]
    goff2 = jnp.arange(G) - gfirst[gp2]
    gvalid1 = (goff1 >= 0) & (goff1 < GACC - 1)
    gtwo = (gp2 > gp1) & (goff2 >= 0) & (goff2 < GACC - 1)
    gmeta = {
        'batch_pad': jnp.concatenate([batch, jnp.zeros((NPG - N,), jnp.int32)]),
        'gi1': gp1 * GACC + jnp.clip(goff1, 0, GACC - 1),
        'gi2': gp2 * GACC + jnp.clip(goff2, 0, GACC - 1),
        'gtwo': gtwo, 'gvalid1': gvalid1,
    }

    deg = jnp.maximum(cnt, 1.0)
    amp = (jnp.log(deg + 1.0) / AVG_LOG)[:, None]
    att = (AVG_LOG / jnp.log(deg + 1.0))[:, None]
    has = (cnt > 0)[:, None]

    x = params['emb'][mol_x] + _mlp2(mol_x_feat, params['feat'], out_norm=True)

    for lyr in params['layers']:
        g1, g2 = _sc_gather2(dst3, src3, x)
        m_edge = _edge_mlp(bond_p, g1, g2, lyr)
        slab = _sc_edge_reduce(ebounds, nbases, m_edge, dst_g)

        outs = []
        for t in range(T):
            flat = slab[t].reshape(NT * NPH * ACC_N, 4 * FT)
            left = flat[ei1]
            right = flat[ei2]
            tw2 = two[:, None]
            v1 = valid1[:, None]
            s = _combine2(left[:, 0:FT], right[:, 0:FT], tw2, v1, jnp.add, 0.0)
            sq = _combine2(left[:, FT:2 * FT], right[:, FT:2 * FT], tw2, v1, jnp.add, 0.0)
            mn_r = _combine2(left[:, 2 * FT:3 * FT], right[:, 2 * FT:3 * FT],
                             tw2, v1, jnp.minimum, 0.0)
            mx_r = _combine2(left[:, 3 * FT:4 * FT], right[:, 3 * FT:4 * FT],
                             tw2, v1, jnp.maximum, 0.0)
            xt = x[:, t * FT:(t + 1) * FT]
            mean = s / deg[:, None]
            sqm = sq / deg[:, None]
            std = jnp.sqrt(jax.nn.relu(sqm - mean ** 2) + 1e-5)
            mn = jnp.where(has, mn_r, 0.0)
            mx = jnp.where(has, mx_r, 0.0)
            aggr = jnp.concatenate([mean, mn, mx, std], axis=-1)
            scaled = jnp.concatenate([aggr, aggr * amp, aggr * att], axis=-1)
            outs.append(_mlp2(jnp.concatenate([xt, scaled], axis=-1), lyr['post'][t]))
        x = jnp.concatenate(outs, axis=-1) @ lyr['lin_W'] + lyr['lin_b']

        mean_g = _graph_segment_sum(x, gbounds, gmeta) / cntb[:, None]
        out = x - lyr['gn_ms'] * mean_g[batch]
        var_g = _graph_segment_sum(out * out, gbounds, gmeta) / cntb[:, None]
        std_g = jnp.sqrt(var_g + 1e-5)
        x = lyr['gn_w'] * out / std_g[batch] + lyr['gn_b']

    pool = _graph_segment_sum(x, gbounds, gmeta) / cntb[:, None]
    return _reg_head(pool, params['reg'])


# speedup vs baseline: 3.0766x; 1.2803x over previous
import functools

import jax
import jax.numpy as jnp
import numpy as np
from jax import lax
from jax.experimental import pallas as pl
from jax.experimental.pallas import tpu as pltpu
from jax.experimental.pallas import tpu_sc as plsc

N = 50000
E = 800000
H = 64
T = 2
FT = H // T
G = 512
MOL_DEG = np.array([0, 1200, 5400, 12000, 18000, 9000, 3200, 1200], dtype=np.float64)
_bins = np.arange(len(MOL_DEG))
AVG_LOG = float((np.log(_bins + 1.0) * MOL_DEG).sum() / MOL_DEG.sum())

NT = 32
CHUNK = 128
CPT = 196
EPAD = NT * CPT * CHUNK
RCH = 128
ACC_N = 449
NPH = 4
GACC = 49
NPG = 50176

_half = [0, 25344, 50688]
while _half[-1] + 24960 < 400000:
    _half.append(_half[-1] + 24960)
EW = np.array(_half + [400000] + [400000 + b for b in _half[1:]] + [800000],
              dtype=np.int32)
GW = np.array([3200 * k for k in range(14)] + [44480, 47360, 50000],
              dtype=np.int32)

_sc_mesh = plsc.VectorSubcoreMesh(core_axis_name="c", subcore_axis_name="s")
_sc_params = pltpu.CompilerParams(use_tc_tiling_on_sc=False)


def _wid():
    return lax.axis_index("s") * 2 + lax.axis_index("c")


@functools.partial(
    pl.kernel,
    mesh=_sc_mesh,
    compiler_params=_sc_params,
    out_type=jax.ShapeDtypeStruct((EPAD, H), jnp.float32),
    scratch_types=[
        pltpu.VMEM((CPT, CHUNK), jnp.int32),
        pltpu.VMEM((3, CHUNK, H), jnp.float32),
        pltpu.SemaphoreType.DMA,
        pltpu.SemaphoreType.DMA((3,)),
        pltpu.SemaphoreType.DMA((3,)),
    ],
)
def _sc_permute(idx_hbm, table_hbm, out_hbm, idx_v, buf, sem_i, sem_g, sem_o):
    w = _wid()
    base = w * (CPT * CHUNK)
    pltpu.async_copy(idx_hbm.at[w], idx_v, sem_i).wait()

    def start_g(j, p):
        pltpu.async_copy(table_hbm.at[idx_v.at[j]], buf.at[p], sem_g.at[p])

    def wait_g(p):
        pltpu.make_async_copy(table_hbm.at[idx_v.at[0]], buf.at[p], sem_g.at[p]).wait()

    def start_o(j, p):
        pltpu.async_copy(buf.at[p], out_hbm.at[pl.ds(base + j * CHUNK, CHUNK), :],
                         sem_o.at[p])

    def wait_o(p):
        pltpu.make_async_copy(buf.at[p], out_hbm.at[pl.ds(0, CHUNK), :],
                              sem_o.at[p]).wait()

    start_g(0, 0)
    start_g(1, 1)

    def body(j, carry):
        p = j % 3

        @pl.when(j >= 1)
        def _():
            wait_o((j - 1) % 3)

        @pl.when(j + 2 < CPT)
        def _():
            start_g(j + 2, (j + 2) % 3)

        wait_g(p)
        start_o(j, p)
        return carry

    lax.fori_loop(0, CPT, body, 0)
    wait_o((CPT - 1) % 3)


@functools.partial(
    pl.kernel,
    mesh=_sc_mesh,
    compiler_params=_sc_params,
    out_type=[jax.ShapeDtypeStruct((EPAD, H), jnp.float32),
              jax.ShapeDtypeStruct((EPAD, H), jnp.float32)],
    scratch_types=[
        pltpu.VMEM((CPT, CHUNK), jnp.int32),
        pltpu.VMEM((CPT, CHUNK), jnp.int32),
        pltpu.VMEM((3, CHUNK, H), jnp.float32),
        pltpu.VMEM((3, CHUNK, H), jnp.float32),
        pltpu.SemaphoreType.DMA,
        pltpu.SemaphoreType.DMA((3,)),
        pltpu.SemaphoreType.DMA((3,)),
        pltpu.SemaphoreType.DMA((3,)),
        pltpu.SemaphoreType.DMA((3,)),
    ],
)
def _sc_gather2(dst_hbm, src_hbm, x_hbm, o1_hbm, o2_hbm,
                di_v, si_v, buf1, buf2, sem_i, sem_a, sem_b, sem_o1, sem_o2):
    w = _wid()
    base = w * (CPT * CHUNK)
    pltpu.async_copy(dst_hbm.at[w], di_v, sem_i).wait()
    pltpu.async_copy(src_hbm.at[w], si_v, sem_i).wait()

    def start_g(j, p):
        pltpu.async_copy(x_hbm.at[di_v.at[j]], buf1.at[p], sem_a.at[p])
        pltpu.async_copy(x_hbm.at[si_v.at[j]], buf2.at[p], sem_b.at[p])

    def wait_g(p):
        pltpu.make_async_copy(x_hbm.at[di_v.at[0]], buf1.at[p], sem_a.at[p]).wait()
        pltpu.make_async_copy(x_hbm.at[si_v.at[0]], buf2.at[p], sem_b.at[p]).wait()

    def start_o(j, p):
        sl = out_slice = pl.ds(base + j * CHUNK, CHUNK)
        pltpu.async_copy(buf1.at[p], o1_hbm.at[sl, :], sem_o1.at[p])
        pltpu.async_copy(buf2.at[p], o2_hbm.at[sl, :], sem_o2.at[p])

    def wait_o(p):
        pltpu.make_async_copy(buf1.at[p], o1_hbm.at[pl.ds(0, CHUNK), :],
                              sem_o1.at[p]).wait()
        pltpu.make_async_copy(buf2.at[p], o2_hbm.at[pl.ds(0, CHUNK), :],
                              sem_o2.at[p]).wait()

    start_g(0, 0)
    start_g(1, 1)

    def body(j, carry):
        p = j % 3

        @pl.when(j >= 1)
        def _():
            wait_o((j - 1) % 3)

        @pl.when(j + 2 < CPT)
        def _():
            start_g(j + 2, (j + 2) % 3)

        wait_g(p)
        start_o(j, p)
        return carry

    lax.fori_loop(0, CPT, body, 0)
    wait_o((CPT - 1) % 3)


@functools.partial(
    pl.kernel,
    mesh=_sc_mesh,
    compiler_params=_sc_params,
    out_type=jax.ShapeDtypeStruct((T, NT * NPH, ACC_N, 4 * FT), jnp.float32),
    scratch_types=[
        pltpu.VMEM((NT, 16), jnp.int32),
        pltpu.VMEM((NT, 16), jnp.int32),
        pltpu.VMEM((ACC_N, 4 * FT), jnp.float32),
        pltpu.VMEM((2, RCH, FT), jnp.float32),
        pltpu.VMEM((2, RCH), jnp.int32),
        pltpu.SemaphoreType.DMA,
        pltpu.SemaphoreType.DMA((2,)),
        pltpu.SemaphoreType.DMA((2,)),
        pltpu.SemaphoreType.DMA,
    ],
)
def _sc_edge_reduce(eb_hbm, nb_hbm, m_hbm, dst_hbm, out_hbm,
                    eb_v, nb_v, acc, mbuf, dbuf, sem_i, sem_m, sem_d, sem_w):
    w = _wid()
    pltpu.async_copy(eb_hbm, eb_v, sem_i).wait()
    pltpu.async_copy(nb_hbm, nb_v, sem_i).wait()
    ebv = eb_v[w, pl.ds(0, 16)]
    nbv = nb_v[w, pl.ds(0, 16)]
    zeros = jnp.zeros((16,), jnp.float32)
    pinf = jnp.full((16,), jnp.inf, jnp.float32)
    ninf = jnp.full((16,), -jnp.inf, jnp.float32)

    for tw in range(T):
        for ph in range(NPH):
            e0 = ebv[ph]
            e1 = ebv[ph + 1]
            nb = nbv[ph]
            e0a = (e0 // 8) * 8
            nch = (e1 - e0a + (RCH - 1)) // RCH

            def initrow(r, c):
                for k in range(2 * FT // 16):
                    acc[r, pl.ds(k * 16, 16)] = zeros
                for k in range(FT // 16):
                    acc[r, pl.ds(2 * FT + k * 16, 16)] = pinf
                for k in range(FT // 16):
                    acc[r, pl.ds(3 * FT + k * 16, 16)] = ninf
                return c

            lax.fori_loop(0, ACC_N, initrow, 0)

            def start_in(ch, p):
                eb = e0a + ch * RCH
                pltpu.async_copy(
                    m_hbm.at[pl.ds(eb, RCH), pl.ds(tw * FT, FT)],
                    mbuf.at[p], sem_m.at[p])
                pltpu.async_copy(dst_hbm.at[pl.ds(eb, RCH)], dbuf.at[p],
                                 sem_d.at[p])

            def wait_in(p):
                pltpu.make_async_copy(
                    m_hbm.at[pl.ds(0, RCH), pl.ds(0, FT)], mbuf.at[p],
                    sem_m.at[p]).wait()
                pltpu.make_async_copy(dst_hbm.at[pl.ds(0, RCH)], dbuf.at[p],
                                      sem_d.at[p]).wait()

            @pl.when(nch > 0)
            def _():
                start_in(0, 0)

            @pl.when(nch > 1)
            def _():
                start_in(1, 1)

            def chunk(ch, c):
                p = ch % 2
                wait_in(p)
                ebase = e0a + ch * RCH

                def edge16(i, c2):
                    jb = i * 16
                    dvec = dbuf[p, pl.ds(jb, 16)] - nb
                    ge_v = lax.iota(jnp.int32, 16) + (ebase + jb)
                    okv = jnp.logical_and(ge_v >= e0, ge_v < e1)
                    rowv = jnp.where(okv, dvec, ACC_N - 1)
                    for k in range(16):
                        row = rowv[k]
                        for f in range(FT // 16):
                            sl = pl.ds(f * 16, 16)
                            v = mbuf[p, jb + k, sl]
                            acc[row, pl.ds(f * 16, 16)] = (
                                acc[row, pl.ds(f * 16, 16)] + v)
                            acc[row, pl.ds(FT + f * 16, 16)] = (
                                acc[row, pl.ds(FT + f * 16, 16)] + v * v)
                            acc[row, pl.ds(2 * FT + f * 16, 16)] = jnp.minimum(
                                acc[row, pl.ds(2 * FT + f * 16, 16)], v)
                            acc[row, pl.ds(3 * FT + f * 16, 16)] = jnp.maximum(
                                acc[row, pl.ds(3 * FT + f * 16, 16)], v)
                    return c2

                lax.fori_loop(0, RCH // 16, edge16, 0)

                @pl.when(ch + 2 < nch)
                def _():
                    start_in(ch + 2, p)

                return c

            lax.fori_loop(0, nch, chunk, 0)
            pltpu.async_copy(acc, out_hbm.at[tw, w * NPH + ph], sem_w).wait()


@functools.partial(
    pl.kernel,
    mesh=_sc_mesh,
    compiler_params=_sc_params,
    out_type=jax.ShapeDtypeStruct((16, GACC, H), jnp.float32),
    scratch_types=[
        pltpu.VMEM((16, 16), jnp.int32),
        pltpu.VMEM((GACC, H), jnp.float32),
        pltpu.VMEM((2, RCH, H), jnp.float32),
        pltpu.VMEM((2, RCH), jnp.int32),
        pltpu.SemaphoreType.DMA,
        pltpu.SemaphoreType.DMA((2,)),
        pltpu.SemaphoreType.DMA((2,)),
        pltpu.SemaphoreType.DMA,
    ],
)
def _sc_graph_sum(gb_hbm, x_hbm, b_hbm, out_hbm,
                  gb_v, acc, xbuf, bbuf, sem_i, sem_x, sem_b, sem_w):
    w = _wid()

    @pl.when(w < 16)
    def _():
        pltpu.async_copy(gb_hbm, gb_v, sem_i).wait()
        gbv = gb_v[w, pl.ds(0, 16)]
        e0 = gbv[0]
        e1 = gbv[1]
        gb = gbv[2]
        zeros = jnp.zeros((16,), jnp.float32)
        e0a = (e0 // 8) * 8
        nch = (e1 - e0a + (RCH - 1)) // RCH

        def initrow(r, c):
            for k in range(H // 16):
                acc[r, pl.ds(k * 16, 16)] = zeros
            return c

        lax.fori_loop(0, GACC, initrow, 0)

        def start_in(ch, p):
            eb = e0a + ch * RCH
            pltpu.async_copy(x_hbm.at[pl.ds(eb, RCH), :], xbuf.at[p], sem_x.at[p])
            pltpu.async_copy(b_hbm.at[pl.ds(eb, RCH)], bbuf.at[p], sem_b.at[p])

        def wait_in(p):
            pltpu.make_async_copy(x_hbm.at[pl.ds(0, RCH), :], xbuf.at[p],
                                  sem_x.at[p]).wait()
            pltpu.make_async_copy(b_hbm.at[pl.ds(0, RCH)], bbuf.at[p],
                                  sem_b.at[p]).wait()

        @pl.when(nch > 0)
        def _():
            start_in(0, 0)

        @pl.when(nch > 1)
        def _():
            start_in(1, 1)

        def chunk(ch, c):
            p = ch % 2
            wait_in(p)
            ebase = e0a + ch * RCH

            def row16(i, c2):
                jb = i * 16
                bvec = bbuf[p, pl.ds(jb, 16)] - gb
                ge_v = lax.iota(jnp.int32, 16) + (ebase + jb)
                okv = jnp.logical_and(ge_v >= e0, ge_v < e1)
                rowv = jnp.where(okv, bvec, GACC - 1)
                for k in range(16):
                    row = rowv[k]
                    for f in range(H // 16):
                        sl = pl.ds(f * 16, 16)
                        acc[row, sl] = acc[row, sl] + xbuf[p, jb + k, sl]
                return c2

            lax.fori_loop(0, RCH // 16, row16, 0)

            @pl.when(ch + 2 < nch)
            def _():
                start_in(ch + 2, p)

            return c

        lax.fori_loop(0, nch, chunk, 0)
        pltpu.async_copy(acc, out_hbm.at[w], sem_w).wait()


_BT = 1024


def _edge_mlp_kernel(bond_ref, g1_ref, g2_ref, ew_ref, eb_ref,
                     w00_ref, b00_ref, w10_ref, b10_ref,
                     w01_ref, b01_ref, w11_ref, b11_ref, out_ref):
    e = bond_ref[...] @ ew_ref[...] + eb_ref[...]
    outs = []
    for t, (w0, b0, w1, b1) in enumerate((
            (w00_ref, b00_ref, w10_ref, b10_ref),
            (w01_ref, b01_ref, w11_ref, b11_ref))):
        h = jnp.concatenate(
            [g1_ref[:, t * FT:(t + 1) * FT], g2_ref[:, t * FT:(t + 1) * FT], e],
            axis=1)
        pre = jnp.maximum(h @ w0[...] + b0[...], 0.0)
        outs.append(pre @ w1[...] + b1[...])
    out_ref[...] = jnp.concatenate(outs, axis=1)


def _edge_mlp(bond_p, g1, g2, lyr):
    grid = (EPAD // _BT,)
    bspec = pl.BlockSpec((_BT, H), lambda i: (i, 0))
    wspec = lambda shape: pl.BlockSpec(shape, lambda i: (0,) * len(shape))
    return pl.pallas_call(
        _edge_mlp_kernel,
        grid=grid,
        in_specs=[bspec, bspec, bspec,
                  wspec((H, FT)), wspec((1, FT)),
                  wspec((3 * FT, FT)), wspec((1, FT)),
                  wspec((FT, FT)), wspec((1, FT)),
                  wspec((3 * FT, FT)), wspec((1, FT)),
                  wspec((FT, FT)), wspec((1, FT))],
        out_specs=pl.BlockSpec((_BT, H), lambda i: (i, 0)),
        out_shape=jax.ShapeDtypeStruct((EPAD, H), jnp.float32),
    )(bond_p, g1, g2,
      lyr['edge_W'], lyr['edge_b'][None, :],
      lyr['pre'][0]['W0'], lyr['pre'][0]['b0'][None, :],
      lyr['pre'][0]['W1'], lyr['pre'][0]['b1'][None, :],
      lyr['pre'][1]['W0'], lyr['pre'][1]['b0'][None, :],
      lyr['pre'][1]['W1'], lyr['pre'][1]['b1'][None, :])


def _reg_kernel(pool_ref, w0_ref, b0_ref, w1_ref, b1_ref, out_ref):
    h = jnp.maximum(pool_ref[...] @ w0_ref[...] + b0_ref[...], 0.0)
    out_ref[...] = h @ w1_ref[...] + b1_ref[...]


def _reg_head(pool, reg):
    return pl.pallas_call(
        _reg_kernel,
        out_shape=jax.ShapeDtypeStruct((G, 1), jnp.float32),
    )(pool, reg['W0'], reg['b0'][None, :], reg['W1'], reg['b1'][None, :])


def _layer_norm(x, wt, b):
    mu = x.mean(-1, keepdims=True)
    var = ((x - mu) ** 2).mean(-1, keepdims=True)
    return (x - mu) / jnp.sqrt(var + 1e-5) * wt + b


def _mlp2(x, m, out_norm=False):
    x = jax.nn.relu(x @ m['W0'] + m['b0'])
    x = x @ m['W1'] + m['b1']
    if out_norm:
        x = _layer_norm(x, m['ln_w'], m['ln_b'])
    return x


def _combine2(left, right, two, valid1, op, fill):
    both = op(left, right)
    return jnp.where(two, both, jnp.where(valid1, left, fill))


def _graph_segment_sum(x, gbounds, gmeta):
    xp = jnp.concatenate([x, jnp.zeros((NPG - N, H), x.dtype)], axis=0)
    slab = _sc_graph_sum(gbounds, xp, gmeta['batch_pad'])
    flat = slab.reshape(16 * GACC, H)
    left = flat[gmeta['gi1']]
    right = flat[gmeta['gi2']]
    return _combine2(left, right, gmeta['gtwo'][:, None],
                     gmeta['gvalid1'][:, None], jnp.add, 0.0)


def kernel(mol_x, mol_x_feat, bond_x, atom_edge_index, clique_x, clique_edge_index, atom2clique_index, residue_x, residue_evo_x, residue_edge_index, residue_edge_weight, mol_batch, params):
    src = atom_edge_index[0]
    dst = atom_edge_index[1]

    perm = jnp.argsort(dst, stable=True)
    dst_s = dst[perm].astype(jnp.int32)
    src_s = src[perm].astype(jnp.int32)
    cnt = jax.ops.segment_sum(jnp.ones((E,), jnp.float32), dst_s, N)
    row_start = jnp.concatenate(
        [jnp.zeros((1,), jnp.int32),
         jnp.cumsum(cnt).astype(jnp.int32)])

    ew = jnp.asarray(EW)
    nlo = dst_s[ew[:-1]]
    nhi = dst_s[ew[1:] - 1]
    qnodes = []
    for k in range(NPH + 1):
        qnodes.append(nlo + ((nhi + 1 - nlo) * k) // NPH)
    qn = jnp.stack(qnodes, axis=1)
    qe = jnp.clip(row_start[qn], ew[:-1, None], ew[1:, None])
    qe = qe.at[:, 0].set(ew[:-1]).at[:, NPH].set(ew[1:])
    ebounds = jnp.zeros((NT, 16), jnp.int32).at[:, :NPH + 1].set(qe)
    nbases = jnp.zeros((NT, 16), jnp.int32).at[:, :NPH].set(qn[:, :NPH])

    nb_flat = qn[:, :NPH].reshape(-1)
    rs = row_start[:N]
    re = row_start[1:]

    def _win_of(pos):
        h = (pos >= 400000).astype(jnp.int32)
        q = pos - h * 400000
        w_in = jnp.where(q < 25344, 0,
                         jnp.where(q < 50688, 1,
                                   jnp.minimum(2 + (q - 50688) // 24960, 15)))
        return h * 16 + w_in

    def _phase_of(pos):
        w = _win_of(jnp.clip(pos, 0, E - 1))
        ph = ((pos >= qe[w, 1]).astype(jnp.int32)
              + (pos >= qe[w, 2]).astype(jnp.int32)
              + (pos >= qe[w, 3]).astype(jnp.int32))
        return w * NPH + ph

    p1 = jnp.clip(_phase_of(rs), 0, NT * NPH - 1)
    p2 = jnp.clip(_phase_of(re - 1), 0, NT * NPH - 1)
    off1 = jnp.arange(N) - nb_flat[p1]
    off2 = jnp.arange(N) - nb_flat[p2]
    valid1 = (off1 >= 0) & (off1 < ACC_N - 1)
    two = (p2 > p1) & (off2 >= 0) & (off2 < ACC_N - 1)
    ei1 = p1 * ACC_N + jnp.clip(off1, 0, ACC_N - 1)
    ei2 = p2 * ACC_N + jnp.clip(off2, 0, ACC_N - 1)

    dst_g = jnp.concatenate([dst_s, jnp.zeros((EPAD - E,), jnp.int32)])
    src_g = jnp.concatenate([src_s, jnp.zeros((EPAD - E,), jnp.int32)])
    perm_g = jnp.concatenate([perm.astype(jnp.int32), jnp.zeros((EPAD - E,), jnp.int32)])
    dst3 = dst_g.reshape(NT, CPT, CHUNK)
    src3 = src_g.reshape(NT, CPT, CHUNK)
    perm3 = perm_g.reshape(NT, CPT, CHUNK)

    bond_p = _sc_permute(perm3, bond_x)

    batch = mol_batch.astype(jnp.int32)
    row_g = jnp.searchsorted(batch, jnp.arange(G + 1), side='left').astype(jnp.int32)
    cntb = jnp.maximum((row_g[1:] - row_g[:-1]).astype(jnp.float32), 1.0)
    gw = jnp.asarray(GW)
    gfirst = batch[jnp.minimum(gw[:-1], N - 1)]
    gbounds = jnp.zeros((16, 16), jnp.int32)
    gbounds = gbounds.at[:, 0].set(gw[:-1]).at[:, 1].set(gw[1:]).at[:, 2].set(gfirst)
    grs = row_g[:G]
    gre = row_g[1:]
    gp1 = jnp.clip(jnp.searchsorted(gw[:-1], grs, side='right') - 1, 0, 15)
    gp2 = jnp.clip(jnp.searchsorted(gw[:-1], gre - 1, side='right') - 1, 0, 15)
    goff1 = jnp.arange(G) - gfirst[gp1]
    goff2 = jnp.arange(G) - gfirst[gp2]
    gvalid1 = (goff1 >= 0) & (goff1 < GACC - 1)
    gtwo = (gp2 > gp1) & (goff2 >= 0) & (goff2 < GACC - 1)
    gmeta = {
        'batch_pad': jnp.concatenate([batch, jnp.zeros((NPG - N,), jnp.int32)]),
        'gi1': gp1 * GACC + jnp.clip(goff1, 0, GACC - 1),
        'gi2': gp2 * GACC + jnp.clip(goff2, 0, GACC - 1),
        'gtwo': gtwo, 'gvalid1': gvalid1,
    }

    deg = jnp.maximum(cnt, 1.0)
    amp = (jnp.log(deg + 1.0) / AVG_LOG)[:, None]
    att = (AVG_LOG / jnp.log(deg + 1.0))[:, None]
    has = (cnt > 0)[:, None]

    x = params['emb'][mol_x] + _mlp2(mol_x_feat, params['feat'], out_norm=True)

    for lyr in params['layers']:
        g1, g2 = _sc_gather2(dst3, src3, x)
        m_edge = _edge_mlp(bond_p, g1, g2, lyr)
        slab = _sc_edge_reduce(ebounds, nbases, m_edge, dst_g)

        outs = []
        for t in range(T):
            flat = slab[t].reshape(NT * NPH * ACC_N, 4 * FT)
            left = flat[ei1]
            right = flat[ei2]
            tw2 = two[:, None]
            v1 = valid1[:, None]
            s = _combine2(left[:, 0:FT], right[:, 0:FT], tw2, v1, jnp.add, 0.0)
            sq = _combine2(left[:, FT:2 * FT], right[:, FT:2 * FT], tw2, v1, jnp.add, 0.0)
            mn_r = _combine2(left[:, 2 * FT:3 * FT], right[:, 2 * FT:3 * FT],
                             tw2, v1, jnp.minimum, 0.0)
            mx_r = _combine2(left[:, 3 * FT:4 * FT], right[:, 3 * FT:4 * FT],
                             tw2, v1, jnp.maximum, 0.0)
            xt = x[:, t * FT:(t + 1) * FT]
            mean = s / deg[:, None]
            sqm = sq / deg[:, None]
            std = jnp.sqrt(jax.nn.relu(sqm - mean ** 2) + 1e-5)
            mn = jnp.where(has, mn_r, 0.0)
            mx = jnp.where(has, mx_r, 0.0)
            aggr = jnp.concatenate([mean, mn, mx, std], axis=-1)
            scaled = jnp.concatenate([aggr, aggr * amp, aggr * att], axis=-1)
            outs.append(_mlp2(jnp.concatenate([xt, scaled], axis=-1), lyr['post'][t]))
        x = jnp.concatenate(outs, axis=-1) @ lyr['lin_W'] + lyr['lin_b']

        mean_g = _graph_segment_sum(x, gbounds, gmeta) / cntb[:, None]
        out = x - lyr['gn_ms'] * mean_g[batch]
        var_g = _graph_segment_sum(out * out, gbounds, gmeta) / cntb[:, None]
        std_g = jnp.sqrt(var_g + 1e-5)
        x = lyr['gn_w'] * out / std_g[batch] + lyr['gn_b']

    pool = _graph_segment_sum(x, gbounds, gmeta) / cntb[:, None]
    return _reg_head(pool, params['reg'])
